# Initial kernel scaffold; baseline (speedup 1.0000x reference)
import jax
import jax.numpy as jnp
from jax.experimental import pallas as pl


def kernel(x, edge_index, W1, b1, W3, b3):
    raise NotImplementedError("write your pallas kernel here")


# speedup vs baseline: 5.6536x; 5.6536x over previous
import jax
import jax.numpy as jnp
from jax import lax
from jax.experimental import pallas as pl
from jax.experimental.pallas import tpu as pltpu
from jax.experimental.pallas import tpu_sc as plsc

N = 10000
D = 128
E = 320000

NC = 2
NS = 16
NW = NC * NS
BLK = 128
K2 = -(-E // (NW * BLK * 8)) * 8
K1 = K2 * NC
EP = NW * K2 * BLK
NPAD = -(-(N + 8) // (NS * 8)) * NS * 8
RPT = NPAD // NS
BN = 1000


def _sc1_body(x_hbm, srcp, dstp, zr_hbm, ones_hbm, out_hbm,
              src_v, dst_v, rows_v, sh, sem):
    cid = lax.axis_index("c")
    sid = lax.axis_index("s")
    r0 = sid * RPT
    pltpu.sync_copy(zr_hbm.at[pl.ds(r0, RPT)], sh.at[pl.ds(r0, RPT)])

    @pl.when(cid == 1)
    def _():
        pltpu.sync_copy(ones_hbm, rows_v)
    plsc.subcore_barrier()
    e0 = sid * K1 * BLK

    @pl.when(cid == 0)
    def _():
        def step(j, carry):
            pltpu.sync_copy(srcp.at[pl.ds(e0 + j * BLK, BLK)], src_v)
            pltpu.sync_copy(dstp.at[pl.ds(e0 + j * BLK, BLK)], dst_v)
            pltpu.async_copy(x_hbm.at[src_v], rows_v, sem).wait()
            pltpu.sync_copy(rows_v, sh.at[dst_v], add=True)
            return carry
        lax.fori_loop(0, K1, step, 0)

    @pl.when(cid == 1)
    def _():
        def step(j, carry):
            pltpu.sync_copy(dstp.at[pl.ds(e0 + j * BLK, BLK)], dst_v)
            pltpu.sync_copy(rows_v, sh.at[dst_v], add=True)
            return carry
        lax.fori_loop(0, K1, step, 0)

    plsc.subcore_barrier()
    pltpu.sync_copy(sh.at[pl.ds(r0, RPT)], out_hbm.at[cid, pl.ds(r0, RPT)])


def _sc2_body(y_hbm, srcp, dstp, zr_hbm, out_hbm,
              src_v, dst_v, rows_v, sh, sem):
    cid = lax.axis_index("c")
    sid = lax.axis_index("s")
    wid = cid * NS + sid
    r0 = sid * RPT
    pltpu.sync_copy(zr_hbm.at[pl.ds(r0, RPT)], sh.at[pl.ds(r0, RPT)])
    plsc.subcore_barrier()
    e0 = wid * K2 * BLK

    def step(j, carry):
        pltpu.sync_copy(srcp.at[pl.ds(e0 + j * BLK, BLK)], src_v)
        pltpu.sync_copy(dstp.at[pl.ds(e0 + j * BLK, BLK)], dst_v)
        pltpu.async_copy(y_hbm.at[src_v], rows_v, sem).wait()
        pltpu.sync_copy(rows_v, sh.at[dst_v], add=True)
        return carry
    lax.fori_loop(0, K2, step, 0)

    plsc.subcore_barrier()
    pltpu.sync_copy(sh.at[pl.ds(r0, RPT)], out_hbm.at[cid, pl.ds(r0, RPT)])


_sc_mesh = plsc.VectorSubcoreMesh(core_axis_name="c", subcore_axis_name="s")
_sc_scratch = [
    pltpu.VMEM((BLK,), jnp.int32),
    pltpu.VMEM((BLK,), jnp.int32),
    pltpu.VMEM((BLK, D), jnp.float32),
    pltpu.VMEM_SHARED((NPAD, D), jnp.float32),
    pltpu.SemaphoreType.DMA,
]
_sc_out = [jax.ShapeDtypeStruct((NC, NPAD, D), jnp.float32)]

_sc_pass1 = pl.kernel(_sc1_body, out_type=_sc_out, mesh=_sc_mesh,
                      scratch_types=_sc_scratch)
_sc_pass2 = pl.kernel(_sc2_body, out_type=_sc_out, mesh=_sc_mesh,
                      scratch_types=_sc_scratch)


def _tc1_body(x_ref, sp_ref, at_ref, bt_ref, w3t_ref, b1_ref, b3_ref, y_ref):
    sp = sp_ref[...]
    s = sp[0]
    c = sp[1, :, :1]
    xa = jnp.dot(x_ref[...], at_ref[...],
                 preferred_element_type=jnp.float32) + b1_ref[...]
    num = c * xa + jnp.dot(s, bt_ref[...], preferred_element_type=jnp.float32)
    h = num / jnp.maximum(c, 1.0)
    g = 0.5 * h * (1.0 + lax.erf(h * 0.7071067811865476))
    y_ref[...] = jnp.dot(g, w3t_ref[...],
                         preferred_element_type=jnp.float32) + b3_ref[...]


def _tc2_body(tp_ref, sp_ref, o_ref):
    c = sp_ref[1, :, :1]
    tp = tp_ref[...]
    o_ref[...] = (tp[0] + tp[1]) / jnp.maximum(c, 1.0)


_tc1 = pl.pallas_call(
    _tc1_body,
    grid=(N // BN,),
    in_specs=[
        pl.BlockSpec((BN, D), lambda i: (i, 0)),
        pl.BlockSpec((NC, BN, D), lambda i: (0, i, 0)),
        pl.BlockSpec((D, D), lambda i: (0, 0)),
        pl.BlockSpec((D, D), lambda i: (0, 0)),
        pl.BlockSpec((D, D), lambda i: (0, 0)),
        pl.BlockSpec((1, D), lambda i: (0, 0)),
        pl.BlockSpec((1, D), lambda i: (0, 0)),
    ],
    out_specs=pl.BlockSpec((BN, D), lambda i: (i, 0)),
    out_shape=jax.ShapeDtypeStruct((N, D), jnp.float32),
)

_tc2 = pl.pallas_call(
    _tc2_body,
    grid=(N // BN,),
    in_specs=[
        pl.BlockSpec((NC, BN, D), lambda i: (0, i, 0)),
        pl.BlockSpec((NC, BN, D), lambda i: (0, i, 0)),
    ],
    out_specs=pl.BlockSpec((BN, D), lambda i: (i, 0)),
    out_shape=jax.ShapeDtypeStruct((N, D), jnp.float32),
)


@jax.jit
def kernel(x, edge_index, W1, b1, W3, b3):
    src = edge_index[0]
    dst = edge_index[1]
    pad = EP - E
    pad_src = (jnp.arange(pad, dtype=jnp.int32) * 37) % N
    pad_dst = N + (jnp.arange(pad, dtype=jnp.int32) % 8)
    srcp = jnp.concatenate([src, pad_src])
    dstp = jnp.concatenate([dst, pad_dst])
    zr = jnp.zeros((NPAD, D), jnp.float32)
    ones_a = jnp.ones((BLK, D), jnp.float32)

    (acc,) = _sc_pass1(x, srcp, dstp, zr, ones_a)

    at = (W1[:, :D] - W1[:, D:]).T
    bt = W1[:, D:].T
    w3t = W3.T
    y = _tc1(x, acc, at, bt, w3t, b1.reshape(1, D), b3.reshape(1, D))

    (acc2,) = _sc_pass2(y, srcp, dstp, zr)
    return _tc2(acc2, acc)


# speedup vs baseline: 11.7701x; 2.0819x over previous
import jax
import jax.numpy as jnp
from jax import lax
from jax.experimental import pallas as pl
from jax.experimental.pallas import tpu as pltpu
from jax.experimental.pallas import tpu_sc as plsc

N = 10000
D = 128
E = 320000

NC = 2
NS = 16
NW = NC * NS
BLK = 128
K2 = -(-E // (NW * BLK * 8)) * 8
K1 = K2 * NC
EP = NW * K2 * BLK
NB = EP // BLK
CB = 40
NPAD = -(-(N + 8) // (NS * 8)) * NS * 8
RPT = NPAD // NS
BN = 1000


def _gather_scatter_chunks(tbl, srcp, dstp, sh, src_v, dst_v, rows, gs, ss,
                           e0b, nblk):
    for chunk in range(nblk // CB):
        b0 = e0b + chunk * CB
        pltpu.sync_copy(srcp.at[pl.ds(b0, CB)], src_v)
        pltpu.sync_copy(dstp.at[pl.ds(b0, CB)], dst_v)
        sc_d = [None, None]
        g_d = [None, None]
        g_d[0] = pltpu.async_copy(tbl.at[src_v.at[0]], rows[0], gs[0])
        for j in range(CB):
            b = j % 2
            if j + 1 < CB:
                nb = (j + 1) % 2
                if sc_d[nb] is not None:
                    sc_d[nb].wait()
                g_d[nb] = pltpu.async_copy(tbl.at[src_v.at[j + 1]],
                                           rows[nb], gs[nb])
            g_d[b].wait()
            sc_d[b] = pltpu.async_copy(rows[b], sh.at[dst_v.at[j]], ss[b],
                                       add=True)
        sc_d[0].wait()
        sc_d[1].wait()


def _ones_scatter_chunks(dstp, sh, dst_v, ones_v, ss, e0b, nblk):
    for chunk in range(nblk // CB):
        b0 = e0b + chunk * CB
        pltpu.sync_copy(dstp.at[pl.ds(b0, CB)], dst_v)
        descs = [pltpu.async_copy(ones_v, sh.at[dst_v.at[j]], ss[0], add=True)
                 for j in range(CB)]
        for d in descs:
            d.wait()


def _sc1_body(x_hbm, srcp, dstp, zr_hbm, ones_hbm, out_hbm,
              src_v, dst_v, rows0, rows1, sh, g0, g1, s0, s1):
    cid = lax.axis_index("c")
    sid = lax.axis_index("s")
    r0 = sid * RPT
    pltpu.sync_copy(zr_hbm.at[pl.ds(r0, RPT)], sh.at[pl.ds(r0, RPT)])

    @pl.when(cid == 1)
    def _():
        pltpu.sync_copy(ones_hbm, rows0)
    plsc.subcore_barrier()
    e0b = sid * K1

    @pl.when(cid == 0)
    def _():
        _gather_scatter_chunks(x_hbm, srcp, dstp, sh, src_v, dst_v,
                               [rows0, rows1], [g0, g1], [s0, s1], e0b, K1)

    @pl.when(cid == 1)
    def _():
        _ones_scatter_chunks(dstp, sh, dst_v, rows0, [s0], e0b, K1)

    plsc.subcore_barrier()
    pltpu.sync_copy(sh.at[pl.ds(r0, RPT)], out_hbm.at[cid, pl.ds(r0, RPT)])


def _sc2_body(y_hbm, srcp, dstp, zr_hbm, out_hbm,
              src_v, dst_v, rows0, rows1, sh, g0, g1, s0, s1):
    cid = lax.axis_index("c")
    sid = lax.axis_index("s")
    r0 = sid * RPT
    pltpu.sync_copy(zr_hbm.at[pl.ds(r0, RPT)], sh.at[pl.ds(r0, RPT)])
    plsc.subcore_barrier()
    e0b = (cid * NS + sid) * K2
    _gather_scatter_chunks(y_hbm, srcp, dstp, sh, src_v, dst_v,
                           [rows0, rows1], [g0, g1], [s0, s1], e0b, K2)
    plsc.subcore_barrier()
    pltpu.sync_copy(sh.at[pl.ds(r0, RPT)], out_hbm.at[cid, pl.ds(r0, RPT)])


_sc_mesh = plsc.VectorSubcoreMesh(core_axis_name="c", subcore_axis_name="s")
_sc_scratch = [
    pltpu.VMEM((CB, BLK), jnp.int32),
    pltpu.VMEM((CB, BLK), jnp.int32),
    pltpu.VMEM((BLK, D), jnp.float32),
    pltpu.VMEM((BLK, D), jnp.float32),
    pltpu.VMEM_SHARED((NPAD, D), jnp.float32),
    pltpu.SemaphoreType.DMA,
    pltpu.SemaphoreType.DMA,
    pltpu.SemaphoreType.DMA,
    pltpu.SemaphoreType.DMA,
]
_sc_out = [jax.ShapeDtypeStruct((NC, NPAD, D), jnp.float32)]

_sc_pass1 = pl.kernel(_sc1_body, out_type=_sc_out, mesh=_sc_mesh,
                      scratch_types=_sc_scratch)
_sc_pass2 = pl.kernel(_sc2_body, out_type=_sc_out, mesh=_sc_mesh,
                      scratch_types=_sc_scratch)


def _tc1_body(x_ref, sp_ref, at_ref, bt_ref, w3t_ref, b1_ref, b3_ref, y_ref):
    sp = sp_ref[...]
    s = sp[0]
    c = sp[1, :, :1]
    xa = jnp.dot(x_ref[...], at_ref[...],
                 preferred_element_type=jnp.float32) + b1_ref[...]
    num = c * xa + jnp.dot(s, bt_ref[...], preferred_element_type=jnp.float32)
    h = num / jnp.maximum(c, 1.0)
    g = 0.5 * h * (1.0 + lax.erf(h * 0.7071067811865476))
    y_ref[...] = jnp.dot(g, w3t_ref[...],
                         preferred_element_type=jnp.float32) + b3_ref[...]


def _tc2_body(tp_ref, sp_ref, o_ref):
    c = sp_ref[1, :, :1]
    tp = tp_ref[...]
    o_ref[...] = (tp[0] + tp[1]) / jnp.maximum(c, 1.0)


_tc1 = pl.pallas_call(
    _tc1_body,
    grid=(N // BN,),
    in_specs=[
        pl.BlockSpec((BN, D), lambda i: (i, 0)),
        pl.BlockSpec((NC, BN, D), lambda i: (0, i, 0)),
        pl.BlockSpec((D, D), lambda i: (0, 0)),
        pl.BlockSpec((D, D), lambda i: (0, 0)),
        pl.BlockSpec((D, D), lambda i: (0, 0)),
        pl.BlockSpec((1, D), lambda i: (0, 0)),
        pl.BlockSpec((1, D), lambda i: (0, 0)),
    ],
    out_specs=pl.BlockSpec((BN, D), lambda i: (i, 0)),
    out_shape=jax.ShapeDtypeStruct((N, D), jnp.float32),
)

_tc2 = pl.pallas_call(
    _tc2_body,
    grid=(N // BN,),
    in_specs=[
        pl.BlockSpec((NC, BN, D), lambda i: (0, i, 0)),
        pl.BlockSpec((NC, BN, D), lambda i: (0, i, 0)),
    ],
    out_specs=pl.BlockSpec((BN, D), lambda i: (i, 0)),
    out_shape=jax.ShapeDtypeStruct((N, D), jnp.float32),
)


@jax.jit
def kernel(x, edge_index, W1, b1, W3, b3):
    src = edge_index[0]
    dst = edge_index[1]
    pad = EP - E
    pad_src = (jnp.arange(pad, dtype=jnp.int32) * 37) % N
    pad_dst = N + (jnp.arange(pad, dtype=jnp.int32) % 8)
    srcp = jnp.concatenate([src, pad_src]).reshape(NB, BLK)
    dstp = jnp.concatenate([dst, pad_dst]).reshape(NB, BLK)
    zr = jnp.zeros((NPAD, D), jnp.float32)
    ones_a = jnp.ones((BLK, D), jnp.float32)

    (acc,) = _sc_pass1(x, srcp, dstp, zr, ones_a)

    at = (W1[:, :D] - W1[:, D:]).T
    bt = W1[:, D:].T
    w3t = W3.T
    y = _tc1(x, acc, at, bt, w3t, b1.reshape(1, D), b3.reshape(1, D))

    (acc2,) = _sc_pass2(y, srcp, dstp, zr)
    return _tc2(acc2, acc)


# speedup vs baseline: 11.9526x; 1.0155x over previous
import jax
import jax.numpy as jnp
from jax import lax
from jax.experimental import pallas as pl
from jax.experimental.pallas import tpu as pltpu
from jax.experimental.pallas import tpu_sc as plsc

N = 10000
D = 128
E = 320000

NC = 2
NS = 16
NW = NC * NS
BLK = 128
K2 = -(-E // (NW * BLK * 8)) * 8
EP = NW * K2 * BLK
NB = EP // BLK
CB = 40
CB1 = 16
NPAD = -(-(N + 8) // (NS * 8)) * NS * 8
RPT = NPAD // NS
HR = NPAD // 128
BT = 128
GRID = NPAD // BT


def _gather_scatter_block_pipeline(tbl, sh, src_v, dst_v, rowsb, gsem, ssem,
                                   nblk, hist_update):
    sc_d = [None, None]
    g_d = [None, None]
    g_d[0] = pltpu.async_copy(tbl.at[src_v.at[0]], rowsb[0], gsem[0])
    for j in range(nblk):
        b = j % 2
        if j + 1 < nblk:
            nb = (j + 1) % 2
            if sc_d[nb] is not None:
                sc_d[nb].wait()
            g_d[nb] = pltpu.async_copy(tbl.at[src_v.at[j + 1]],
                                       rowsb[nb], gsem[nb])
        g_d[b].wait()
        sc_d[b] = pltpu.async_copy(rowsb[b], sh.at[dst_v.at[j]], ssem[b],
                                   add=True)
        if hist_update is not None:
            hist_update(j)
    sc_d[0].wait()
    sc_d[1].wait()


def _sc1_body(x_hbm, srcp, dstp, zr_hbm, zc_hbm, iota_hbm,
              acc_out, cnt_out,
              src_v, dst_v, rows0, rows1, hist, iota_v, sh, csh,
              g0, g1, s0, s1):
    cid = lax.axis_index("c")
    sid = lax.axis_index("s")
    wid = cid * NS + sid
    r0 = sid * RPT
    pltpu.sync_copy(zr_hbm.at[pl.ds(r0, RPT)], sh.at[pl.ds(r0, RPT)])

    @pl.when(sid == 0)
    def _():
        pltpu.sync_copy(zc_hbm, csh)
    pltpu.sync_copy(iota_hbm, iota_v)

    def zrow(i, carry):
        def zcol(j, c2):
            hist[i, pl.ds(j * 16, 16)] = jnp.zeros((16,), jnp.float32)
            return c2
        return lax.fori_loop(0, D // 16, zcol, carry)
    lax.fori_loop(0, HR, zrow, 0)
    plsc.subcore_barrier()

    def hist_update(j):
        for v8 in range(BLK // 16):
            d16 = dst_v[j, pl.ds(v8 * 16, 16)]
            cnt, lastm = plsc.scan_count(d16)
            plsc.addupdate_scatter(
                hist, (lax.shift_right_logical(d16, 7),
                       lax.bitwise_and(d16, 127)),
                cnt.astype(jnp.float32), mask=lastm)

    e0b = wid * K2
    for chunk in range(K2 // CB1):
        b0 = e0b + chunk * CB1
        pltpu.sync_copy(srcp.at[pl.ds(b0, CB1)], src_v)
        pltpu.sync_copy(dstp.at[pl.ds(b0, CB1)], dst_v)
        _gather_scatter_block_pipeline(x_hbm, sh, src_v, dst_v,
                                       [rows0, rows1], [g0, g1], [s0, s1],
                                       CB1, hist_update)
    pltpu.sync_copy(hist, csh.at[iota_v], add=True)
    plsc.subcore_barrier()
    pltpu.sync_copy(sh.at[pl.ds(r0, RPT)], acc_out.at[cid, pl.ds(r0, RPT)])

    @pl.when(sid == 0)
    def _():
        pltpu.sync_copy(csh, cnt_out.at[cid])


def _sc2_body(y_hbm, srcp, dstp, zr_hbm, out_hbm,
              src_v, dst_v, rows0, rows1, sh, g0, g1, s0, s1):
    cid = lax.axis_index("c")
    sid = lax.axis_index("s")
    r0 = sid * RPT
    pltpu.sync_copy(zr_hbm.at[pl.ds(r0, RPT)], sh.at[pl.ds(r0, RPT)])
    plsc.subcore_barrier()
    e0b = (cid * NS + sid) * K2
    for chunk in range(K2 // CB):
        b0 = e0b + chunk * CB
        pltpu.sync_copy(srcp.at[pl.ds(b0, CB)], src_v)
        pltpu.sync_copy(dstp.at[pl.ds(b0, CB)], dst_v)
        _gather_scatter_block_pipeline(y_hbm, sh, src_v, dst_v,
                                       [rows0, rows1], [g0, g1], [s0, s1],
                                       CB, None)
    plsc.subcore_barrier()
    pltpu.sync_copy(sh.at[pl.ds(r0, RPT)], out_hbm.at[cid, pl.ds(r0, RPT)])


_sc_mesh = plsc.VectorSubcoreMesh(core_axis_name="c", subcore_axis_name="s")

_sc_pass1 = pl.kernel(
    _sc1_body,
    out_type=[jax.ShapeDtypeStruct((NC, NPAD, D), jnp.float32),
              jax.ShapeDtypeStruct((NC, HR, D), jnp.float32)],
    mesh=_sc_mesh,
    compiler_params=pltpu.CompilerParams(needs_layout_passes=False),
    scratch_types=[
        pltpu.VMEM((CB1, BLK), jnp.int32),
        pltpu.VMEM((CB1, BLK), jnp.int32),
        pltpu.VMEM((BLK, D), jnp.float32),
        pltpu.VMEM((BLK, D), jnp.float32),
        pltpu.VMEM((HR, D), jnp.float32),
        pltpu.VMEM((HR,), jnp.int32),
        pltpu.VMEM_SHARED((NPAD, D), jnp.float32),
        pltpu.VMEM_SHARED((HR, D), jnp.float32),
        pltpu.SemaphoreType.DMA,
        pltpu.SemaphoreType.DMA,
        pltpu.SemaphoreType.DMA,
        pltpu.SemaphoreType.DMA,
    ])

_sc_pass2 = pl.kernel(
    _sc2_body,
    out_type=[jax.ShapeDtypeStruct((NC, NPAD, D), jnp.float32)],
    mesh=_sc_mesh,
    scratch_types=[
        pltpu.VMEM((CB, BLK), jnp.int32),
        pltpu.VMEM((CB, BLK), jnp.int32),
        pltpu.VMEM((BLK, D), jnp.float32),
        pltpu.VMEM((BLK, D), jnp.float32),
        pltpu.VMEM_SHARED((NPAD, D), jnp.float32),
        pltpu.SemaphoreType.DMA,
        pltpu.SemaphoreType.DMA,
        pltpu.SemaphoreType.DMA,
        pltpu.SemaphoreType.DMA,
    ])


def _tc1_body(x_ref, sp_ref, cnt_ref, at_ref, bt_ref, w3t_ref, b1_ref, b3_ref,
              y_ref):
    sp = sp_ref[...]
    s = sp[0] + sp[1]
    c = (cnt_ref[0, 0, 0] + cnt_ref[0, 0, 1])[:, None]
    xa = jnp.dot(x_ref[...], at_ref[...],
                 preferred_element_type=jnp.float32) + b1_ref[...]
    num = c * xa + jnp.dot(s, bt_ref[...], preferred_element_type=jnp.float32)
    h = num / jnp.maximum(c, 1.0)
    g = 0.5 * h * (1.0 + lax.erf(h * 0.7071067811865476))
    y_ref[...] = jnp.dot(g, w3t_ref[...],
                         preferred_element_type=jnp.float32) + b3_ref[...]


def _tc2_body(tp_ref, cnt_ref, o_ref):
    c = (cnt_ref[0, 0, 0] + cnt_ref[0, 0, 1])[:, None]
    tp = tp_ref[...]
    o_ref[...] = (tp[0] + tp[1]) / jnp.maximum(c, 1.0)


_tc1 = pl.pallas_call(
    _tc1_body,
    grid=(GRID,),
    in_specs=[
        pl.BlockSpec((BT, D), lambda i: (i, 0)),
        pl.BlockSpec((NC, BT, D), lambda i: (0, i, 0)),
        pl.BlockSpec((1, 1, NC, D), lambda i: (i, 0, 0, 0)),
        pl.BlockSpec((D, D), lambda i: (0, 0)),
        pl.BlockSpec((D, D), lambda i: (0, 0)),
        pl.BlockSpec((D, D), lambda i: (0, 0)),
        pl.BlockSpec((1, D), lambda i: (0, 0)),
        pl.BlockSpec((1, D), lambda i: (0, 0)),
    ],
    out_specs=pl.BlockSpec((BT, D), lambda i: (i, 0)),
    out_shape=jax.ShapeDtypeStruct((N, D), jnp.float32),
)

_tc2 = pl.pallas_call(
    _tc2_body,
    grid=(GRID,),
    in_specs=[
        pl.BlockSpec((NC, BT, D), lambda i: (0, i, 0)),
        pl.BlockSpec((1, 1, NC, D), lambda i: (i, 0, 0, 0)),
    ],
    out_specs=pl.BlockSpec((BT, D), lambda i: (i, 0)),
    out_shape=jax.ShapeDtypeStruct((N, D), jnp.float32),
)


@jax.jit
def kernel(x, edge_index, W1, b1, W3, b3):
    src = edge_index[0]
    dst = edge_index[1]
    pad = EP - E
    pad_src = (jnp.arange(pad, dtype=jnp.int32) * 37) % N
    pad_dst = N + (jnp.arange(pad, dtype=jnp.int32) % 8)
    srcp = jnp.concatenate([src, pad_src]).reshape(NB, BLK)
    dstp = jnp.concatenate([dst, pad_dst]).reshape(NB, BLK)
    zr = jnp.zeros((NPAD, D), jnp.float32)
    zc = jnp.zeros((HR, D), jnp.float32)
    iota = jnp.arange(HR, dtype=jnp.int32)

    acc, cnt = _sc_pass1(x, srcp, dstp, zr, zc, iota)
    cntt = cnt.transpose(1, 0, 2).reshape(HR, 1, NC, D)

    at = (W1[:, :D] - W1[:, D:]).T
    bt = W1[:, D:].T
    w3t = W3.T
    y = _tc1(x, acc, cntt, at, bt, w3t, b1.reshape(1, D), b3.reshape(1, D))

    (acc2,) = _sc_pass2(y, srcp, dstp, zr)
    return _tc2(acc2, cntt)


# speedup vs baseline: 14.9261x; 1.2488x over previous
import jax
import jax.numpy as jnp
from jax import lax
from jax.experimental import pallas as pl
from jax.experimental.pallas import tpu as pltpu
from jax.experimental.pallas import tpu_sc as plsc

N = 10000
D = 128
E = 320000

NC = 2
NS = 16
NW = NC * NS
BLK = 128
K2 = -(-E // (NW * BLK * 8)) * 8
EP = NW * K2 * BLK
NB = EP // BLK
CB = 40
CB1 = 16
NPAD = -(-(N + 8) // (NS * 8)) * NS * 8
RPT = NPAD // NS
HR = NPAD // 128
BT = 1024
GRID = -(-N // BT)
CR = BT // 128
HRP = GRID * CR


def _gather_scatter_block_pipeline(tbl, sh, src_v, dst_v, rowsb, gsem, ssem,
                                   nblk, hist_update):
    sc_d = [None, None]
    g_d = [None, None]
    g_d[0] = pltpu.async_copy(tbl.at[src_v.at[0]], rowsb[0], gsem[0])
    for j in range(nblk):
        b = j % 2
        if j + 1 < nblk:
            nb = (j + 1) % 2
            if sc_d[nb] is not None:
                sc_d[nb].wait()
            g_d[nb] = pltpu.async_copy(tbl.at[src_v.at[j + 1]],
                                       rowsb[nb], gsem[nb])
        g_d[b].wait()
        sc_d[b] = pltpu.async_copy(rowsb[b], sh.at[dst_v.at[j]], ssem[b],
                                   add=True)
        if hist_update is not None:
            hist_update(j)
    sc_d[0].wait()
    sc_d[1].wait()


def _sc1_body(x_hbm, srcp, dstp, zr_hbm, zc_hbm, iota_hbm,
              acc_out, cnt_out,
              src_v, dst_v, rows0, rows1, hist, iota_v, sh, csh,
              g0, g1, s0, s1):
    cid = lax.axis_index("c")
    sid = lax.axis_index("s")
    wid = cid * NS + sid
    r0 = sid * RPT
    pltpu.sync_copy(zr_hbm.at[pl.ds(r0, RPT)], sh.at[pl.ds(r0, RPT)])

    @pl.when(sid == 0)
    def _():
        pltpu.sync_copy(zc_hbm, csh)
    pltpu.sync_copy(iota_hbm, iota_v)

    def zrow(i, carry):
        def zcol(j, c2):
            hist[i, pl.ds(j * 16, 16)] = jnp.zeros((16,), jnp.float32)
            return c2
        return lax.fori_loop(0, D // 16, zcol, carry)
    lax.fori_loop(0, HR, zrow, 0)
    plsc.subcore_barrier()

    def hist_update(j):
        for v8 in range(BLK // 16):
            d16 = dst_v[j, pl.ds(v8 * 16, 16)]
            cnt, lastm = plsc.scan_count(d16)
            plsc.addupdate_scatter(
                hist, (lax.shift_right_logical(d16, 7),
                       lax.bitwise_and(d16, 127)),
                cnt.astype(jnp.float32), mask=lastm)

    e0b = wid * K2
    for chunk in range(K2 // CB1):
        b0 = e0b + chunk * CB1
        pltpu.sync_copy(srcp.at[pl.ds(b0, CB1)], src_v)
        pltpu.sync_copy(dstp.at[pl.ds(b0, CB1)], dst_v)
        _gather_scatter_block_pipeline(x_hbm, sh, src_v, dst_v,
                                       [rows0, rows1], [g0, g1], [s0, s1],
                                       CB1, hist_update)
    pltpu.sync_copy(hist, csh.at[iota_v], add=True)
    plsc.subcore_barrier()
    pltpu.sync_copy(sh.at[pl.ds(r0, RPT)], acc_out.at[cid, pl.ds(r0, RPT)])

    @pl.when(sid == 0)
    def _():
        pltpu.sync_copy(csh, cnt_out.at[cid])


def _sc2_body(y_hbm, srcp, dstp, zr_hbm, out_hbm,
              src_v, dst_v, rows0, rows1, sh, g0, g1, s0, s1):
    cid = lax.axis_index("c")
    sid = lax.axis_index("s")
    r0 = sid * RPT
    pltpu.sync_copy(zr_hbm.at[pl.ds(r0, RPT)], sh.at[pl.ds(r0, RPT)])
    plsc.subcore_barrier()
    e0b = (cid * NS + sid) * K2
    for chunk in range(K2 // CB):
        b0 = e0b + chunk * CB
        pltpu.sync_copy(srcp.at[pl.ds(b0, CB)], src_v)
        pltpu.sync_copy(dstp.at[pl.ds(b0, CB)], dst_v)
        _gather_scatter_block_pipeline(y_hbm, sh, src_v, dst_v,
                                       [rows0, rows1], [g0, g1], [s0, s1],
                                       CB, None)
    plsc.subcore_barrier()
    pltpu.sync_copy(sh.at[pl.ds(r0, RPT)], out_hbm.at[cid, pl.ds(r0, RPT)])


_sc_mesh = plsc.VectorSubcoreMesh(core_axis_name="c", subcore_axis_name="s")

_sc_pass1 = pl.kernel(
    _sc1_body,
    out_type=[jax.ShapeDtypeStruct((NC, NPAD, D), jnp.float32),
              jax.ShapeDtypeStruct((NC, HR, D), jnp.float32)],
    mesh=_sc_mesh,
    compiler_params=pltpu.CompilerParams(needs_layout_passes=False),
    scratch_types=[
        pltpu.VMEM((CB1, BLK), jnp.int32),
        pltpu.VMEM((CB1, BLK), jnp.int32),
        pltpu.VMEM((BLK, D), jnp.float32),
        pltpu.VMEM((BLK, D), jnp.float32),
        pltpu.VMEM((HR, D), jnp.float32),
        pltpu.VMEM((HR,), jnp.int32),
        pltpu.VMEM_SHARED((NPAD, D), jnp.float32),
        pltpu.VMEM_SHARED((HR, D), jnp.float32),
        pltpu.SemaphoreType.DMA,
        pltpu.SemaphoreType.DMA,
        pltpu.SemaphoreType.DMA,
        pltpu.SemaphoreType.DMA,
    ])

_sc_pass2 = pl.kernel(
    _sc2_body,
    out_type=[jax.ShapeDtypeStruct((NC, NPAD, D), jnp.float32)],
    mesh=_sc_mesh,
    scratch_types=[
        pltpu.VMEM((CB, BLK), jnp.int32),
        pltpu.VMEM((CB, BLK), jnp.int32),
        pltpu.VMEM((BLK, D), jnp.float32),
        pltpu.VMEM((BLK, D), jnp.float32),
        pltpu.VMEM_SHARED((NPAD, D), jnp.float32),
        pltpu.SemaphoreType.DMA,
        pltpu.SemaphoreType.DMA,
        pltpu.SemaphoreType.DMA,
        pltpu.SemaphoreType.DMA,
    ])


def _block_counts(cnt_ref):
    cs = cnt_ref[...]
    c = cs[:, 0] + cs[:, 1]
    return [c[r][:, None] for r in range(CR)]


def _tc1_body(x_ref, sp_ref, cnt_ref, at_ref, bt_ref, w3t_ref, b1_ref, b3_ref,
              y_ref):
    sp = sp_ref[...]
    s = sp[0] + sp[1]
    xa = jnp.dot(x_ref[...], at_ref[...],
                 preferred_element_type=jnp.float32) + b1_ref[...]
    sb = jnp.dot(s, bt_ref[...], preferred_element_type=jnp.float32)
    cr = _block_counts(cnt_ref)
    for r in range(CR):
        c = cr[r]
        lo, hi = r * 128, (r + 1) * 128
        h = (c * xa[lo:hi] + sb[lo:hi]) / jnp.maximum(c, 1.0)
        g = 0.5 * h * (1.0 + lax.erf(h * 0.7071067811865476))
        y_ref[pl.ds(lo, 128), :] = jnp.dot(
            g, w3t_ref[...], preferred_element_type=jnp.float32) + b3_ref[...]


def _tc2_body(tp_ref, cnt_ref, o_ref):
    tp = tp_ref[...]
    t = tp[0] + tp[1]
    cr = _block_counts(cnt_ref)
    for r in range(CR):
        lo = r * 128
        o_ref[pl.ds(lo, 128), :] = t[lo:lo + 128] / jnp.maximum(cr[r], 1.0)


_tc1 = pl.pallas_call(
    _tc1_body,
    grid=(GRID,),
    in_specs=[
        pl.BlockSpec((BT, D), lambda i: (i, 0)),
        pl.BlockSpec((NC, BT, D), lambda i: (0, i, 0)),
        pl.BlockSpec((CR, NC, D), lambda i: (i, 0, 0)),
        pl.BlockSpec((D, D), lambda i: (0, 0)),
        pl.BlockSpec((D, D), lambda i: (0, 0)),
        pl.BlockSpec((D, D), lambda i: (0, 0)),
        pl.BlockSpec((1, D), lambda i: (0, 0)),
        pl.BlockSpec((1, D), lambda i: (0, 0)),
    ],
    out_specs=pl.BlockSpec((BT, D), lambda i: (i, 0)),
    out_shape=jax.ShapeDtypeStruct((N, D), jnp.float32),
)

_tc2 = pl.pallas_call(
    _tc2_body,
    grid=(GRID,),
    in_specs=[
        pl.BlockSpec((NC, BT, D), lambda i: (0, i, 0)),
        pl.BlockSpec((CR, NC, D), lambda i: (i, 0, 0)),
    ],
    out_specs=pl.BlockSpec((BT, D), lambda i: (i, 0)),
    out_shape=jax.ShapeDtypeStruct((N, D), jnp.float32),
)


@jax.jit
def kernel(x, edge_index, W1, b1, W3, b3):
    src = edge_index[0]
    dst = edge_index[1]
    pad = EP - E
    pad_src = (jnp.arange(pad, dtype=jnp.int32) * 37) % N
    pad_dst = N + (jnp.arange(pad, dtype=jnp.int32) % 8)
    srcp = jnp.concatenate([src, pad_src]).reshape(NB, BLK)
    dstp = jnp.concatenate([dst, pad_dst]).reshape(NB, BLK)
    zr = jnp.zeros((NPAD, D), jnp.float32)
    zc = jnp.zeros((HR, D), jnp.float32)
    iota = jnp.arange(HR, dtype=jnp.int32)

    acc, cnt = _sc_pass1(x, srcp, dstp, zr, zc, iota)
    cntt = jnp.pad(cnt.transpose(1, 0, 2), ((0, HRP - HR), (0, 0), (0, 0)))

    at = (W1[:, :D] - W1[:, D:]).T
    bt = W1[:, D:].T
    w3t = W3.T
    y = _tc1(x, acc, cntt, at, bt, w3t, b1.reshape(1, D), b3.reshape(1, D))

    (acc2,) = _sc_pass2(y, srcp, dstp, zr)
    return _tc2(acc2, cntt)
